# baseline (speedup 1.0000x reference)
import functools

import jax
import jax.numpy as jnp
from jax import lax
from jax.experimental import pallas as pl
from jax.experimental.pallas import tpu as pltpu
from jax.experimental.pallas import tpu_sc as plsc

VOCAB = 1000000
DIM = 64
BATCH = 16384
SEQ = 50

NC = 2
NS = 16
NW = NC * NS
BB = BATCH // 128
UNITS = SEQ * BB
PER_W = UNITS // NW
CG = DIM // 8
OCT = PER_W // 8

_mesh = plsc.VectorSubcoreMesh(core_axis_name="c", subcore_axis_name="s")


@functools.partial(
    pl.kernel,
    mesh=_mesh,
    compiler_params=pltpu.CompilerParams(
        use_tc_tiling_on_sc=False, needs_layout_passes=False),
    out_type=jax.ShapeDtypeStruct((SEQ, CG, BB, 1024), jnp.float32),
    scratch_types=[
        pltpu.VMEM((8, 128), jnp.int32),
        pltpu.VMEM((4, 128, DIM), jnp.float32),
        pltpu.VMEM((4, DIM * 128), jnp.float32),
        pltpu.SemaphoreType.DMA((8,)),
        pltpu.SemaphoreType.DMA((4,)),
        pltpu.SemaphoreType.DMA((4,)),
    ],
)
def _gather(xt_hbm, table_hbm, out_hbm, idx_v, buf_v, tbuf_v, s_idx, s_g, s_wb):
    wid = lax.axis_index("s") * NC + lax.axis_index("c")
    u0 = wid * PER_W

    def fire_idx(u, m):
        s = u // BB
        bb = u - s * BB
        pltpu.async_copy(
            xt_hbm.at[s, pl.ds(pl.multiple_of(bb * 128, 128), 128)],
            idx_v.at[m], s_idx.at[m])

    def wait_idx(m):
        pltpu.make_async_copy(
            xt_hbm.at[0, pl.ds(0, 128)], idx_v.at[m], s_idx.at[m]).wait()

    def fire_gather(m, b):
        pltpu.async_copy(table_hbm.at[idx_v.at[m]], buf_v.at[b], s_g.at[b])

    def wait_gather(b):
        pltpu.make_async_copy(
            table_hbm.at[pl.ds(0, 128)], buf_v.at[b], s_g.at[b]).wait()

    rows_tab = [lax.iota(jnp.int32, 16) + (jb * 16) for jb in range(8)]
    zeros16 = jnp.full((16,), 0, jnp.int32)

    def transpose(b):
        buf = buf_v.at[b]

        def col(c, carry):
            cols = zeros16 + c
            base = c * 128
            for jb in range(8):
                v = plsc.load_gather(buf, [rows_tab[jb], cols])
                tbuf_v[b, pl.ds(base + jb * 16, 16)] = v
            return carry

        lax.fori_loop(0, DIM, col, 0, unroll=2)

    def start_wb(u, b):
        s = u // BB
        bb = u - s * BB
        for cg in range(CG):
            pltpu.async_copy(
                tbuf_v.at[b, pl.ds(cg * 1024, 1024)],
                out_hbm.at[s, cg, bb], s_wb.at[b])

    def wait_wb(b):
        for cg in range(CG):
            pltpu.make_async_copy(
                tbuf_v.at[b, pl.ds(cg * 1024, 1024)],
                out_hbm.at[0, 0, 0], s_wb.at[b]).wait()

    def step(i, k, do_wait_wb, do_fire_idx, do_fire_gather):
        b = k % 4
        wait_gather(b)
        if do_wait_wb:
            wait_wb(b)
        transpose(b)
        start_wb(i, b)
        if do_fire_idx:
            fire_idx(i + 8, k)
        if do_fire_gather:
            wait_idx((k + 3) % 8)
            fire_gather((k + 3) % 8, (k + 3) % 4)

    for m in range(8):
        fire_idx(u0 + m, m)
    for j in range(3):
        wait_idx(j)
        fire_gather(j, j)
    for k in range(8):
        step(u0 + k, k, do_wait_wb=(k >= 4), do_fire_idx=True,
             do_fire_gather=True)

    def octet(q, carry):
        ub = u0 + q * 8
        for k in range(8):
            step(ub + k, k, do_wait_wb=True, do_fire_idx=True,
                 do_fire_gather=True)
        return carry

    lax.fori_loop(1, OCT - 1, octet, 0)

    ub = u0 + (OCT - 1) * 8
    for k in range(8):
        step(ub + k, k, do_wait_wb=True, do_fire_idx=False,
             do_fire_gather=(k <= 4))
    for b in range(4):
        wait_wb(b)


def kernel(x, table):
    xt = x.T
    lout = _gather(xt, table)
    lout5 = lout.reshape(SEQ, CG, BB, 8, 128)
    return lout5.transpose(2, 4, 0, 1, 3).reshape(BATCH, SEQ, DIM)


# speedup vs baseline: 1.6091x; 1.6091x over previous
import functools

import jax
import jax.numpy as jnp
from jax import lax
from jax.experimental import pallas as pl
from jax.experimental.pallas import tpu as pltpu
from jax.experimental.pallas import tpu_sc as plsc

VOCAB = 1000000
DIM = 64
BATCH = 16384
SEQ = 50

NC = 2
NS = 16
NW = NC * NS
BB = BATCH // 128
UNITS = SEQ * BB
PER_W = UNITS // NW
CG = DIM // 8
OCT = PER_W // 8

_mesh = plsc.VectorSubcoreMesh(core_axis_name="c", subcore_axis_name="s")


@functools.partial(
    pl.kernel,
    mesh=_mesh,
    compiler_params=pltpu.CompilerParams(
        use_tc_tiling_on_sc=False, needs_layout_passes=False),
    out_type=jax.ShapeDtypeStruct((SEQ, CG, BB, 8, 128), jnp.float32),
    scratch_types=[
        pltpu.VMEM((8, 128), jnp.int32),
        pltpu.VMEM((4, 128, DIM), jnp.float32),
        pltpu.VMEM((4, DIM, 137), jnp.float32),
        pltpu.SemaphoreType.DMA((8,)),
        pltpu.SemaphoreType.DMA((4,)),
        pltpu.SemaphoreType.DMA((4,)),
    ],
)
def _gather(xt_hbm, table_hbm, out_hbm, idx_v, buf_v, tbuf_v, s_idx, s_g, s_wb):
    wid = lax.axis_index("s") * NC + lax.axis_index("c")
    u0 = wid * PER_W

    def fire_idx(u, m):
        s = u // BB
        bb = u - s * BB
        pltpu.async_copy(
            xt_hbm.at[s, pl.ds(pl.multiple_of(bb * 128, 128), 128)],
            idx_v.at[m], s_idx.at[m])

    def wait_idx(m):
        pltpu.make_async_copy(
            xt_hbm.at[0, pl.ds(0, 128)], idx_v.at[m], s_idx.at[m]).wait()

    def fire_gather(m, b):
        pltpu.async_copy(table_hbm.at[idx_v.at[m]], buf_v.at[b], s_g.at[b])

    def wait_gather(b):
        pltpu.make_async_copy(
            table_hbm.at[pl.ds(0, 128)], buf_v.at[b], s_g.at[b]).wait()

    cols_tab = [lax.iota(jnp.int32, 16) + (cc * 16) for cc in range(4)]
    zeros16 = jnp.full((16,), 0, jnp.int32)

    def transpose(b):
        buf = buf_v.at[b]
        tbuf = tbuf_v.at[b]

        def row(j, carry):
            jv = zeros16 + j
            for cc in range(4):
                v = buf[j, pl.ds(cc * 16, 16)]
                plsc.store_scatter(tbuf, [cols_tab[cc], jv], v)
            return carry

        lax.fori_loop(0, 128, row, 0, unroll=4)

    def start_wb(u, b):
        s = u // BB
        bb = u - s * BB
        for cg in range(CG):
            pltpu.async_copy(
                tbuf_v.at[b, pl.ds(cg * 8, 8), pl.ds(0, 128)],
                out_hbm.at[s, cg, bb], s_wb.at[b])

    def wait_wb(b):
        for cg in range(CG):
            pltpu.make_async_copy(
                tbuf_v.at[b, pl.ds(cg * 8, 8), pl.ds(0, 128)],
                out_hbm.at[0, 0, 0], s_wb.at[b]).wait()

    def step(i, k, do_wait_wb, do_fire_idx, do_fire_gather):
        b = k % 4
        wait_gather(b)
        if do_wait_wb:
            wait_wb(b)
        transpose(b)
        start_wb(i, b)
        if do_fire_idx:
            fire_idx(i + 8, k)
        if do_fire_gather:
            wait_idx((k + 3) % 8)
            fire_gather((k + 3) % 8, (k + 3) % 4)

    for m in range(8):
        fire_idx(u0 + m, m)
    for j in range(3):
        wait_idx(j)
        fire_gather(j, j)
    for k in range(8):
        step(u0 + k, k, do_wait_wb=(k >= 4), do_fire_idx=True,
             do_fire_gather=True)

    def octet(q, carry):
        ub = u0 + q * 8
        for k in range(8):
            step(ub + k, k, do_wait_wb=True, do_fire_idx=True,
                 do_fire_gather=True)
        return carry

    lax.fori_loop(1, OCT - 1, octet, 0)

    ub = u0 + (OCT - 1) * 8
    for k in range(8):
        step(ub + k, k, do_wait_wb=True, do_fire_idx=False,
             do_fire_gather=(k <= 4))
    for b in range(4):
        wait_wb(b)


def kernel(x, table):
    xt = x.T
    lout = _gather(xt, table)
    return lout.transpose(2, 4, 0, 1, 3).reshape(BATCH, SEQ, DIM)


# speedup vs baseline: 1.7030x; 1.0584x over previous
import functools

import jax
import jax.numpy as jnp
from jax import lax
from jax.experimental import pallas as pl
from jax.experimental.pallas import tpu as pltpu
from jax.experimental.pallas import tpu_sc as plsc

VOCAB = 1000000
DIM = 64
BATCH = 16384
SEQ = 50

NC = 2
NS = 16
NW = NC * NS
BB = BATCH // 128
UNITS = SEQ * BB
PER_W = UNITS // NW
CG = DIM // 8
OCT = PER_W // 8

_mesh = plsc.VectorSubcoreMesh(core_axis_name="c", subcore_axis_name="s")


@functools.partial(
    pl.kernel,
    mesh=_mesh,
    compiler_params=pltpu.CompilerParams(
        use_tc_tiling_on_sc=False, needs_layout_passes=False),
    out_type=jax.ShapeDtypeStruct((SEQ, CG, BB, 8, 128), jnp.float32),
    scratch_types=[
        pltpu.VMEM((8, 128), jnp.int32),
        pltpu.VMEM((4, 128, 128), jnp.float32),
        pltpu.VMEM((4, DIM, 137), jnp.float32),
        pltpu.SemaphoreType.DMA((8,)),
        pltpu.SemaphoreType.DMA((4,)),
        pltpu.SemaphoreType.DMA((4,)),
    ],
)
def _gather(xt_hbm, table_hbm, out_hbm, idx_v, buf_v, tbuf_v, s_idx, s_g, s_wb):
    wid = lax.axis_index("s") * NC + lax.axis_index("c")
    u0 = wid * PER_W

    def fire_idx(u, m):
        s = u // BB
        bb = u - s * BB
        pltpu.async_copy(
            xt_hbm.at[s, pl.ds(pl.multiple_of(bb * 128, 128), 128)],
            idx_v.at[m], s_idx.at[m])

    def wait_idx(m):
        pltpu.make_async_copy(
            xt_hbm.at[0, pl.ds(0, 128)], idx_v.at[m], s_idx.at[m]).wait()

    def fire_gather(m, b):
        pltpu.async_copy(table_hbm.at[idx_v.at[m]], buf_v.at[b], s_g.at[b])

    def wait_gather(b):
        pltpu.make_async_copy(
            table_hbm.at[pl.ds(0, 128)], buf_v.at[b], s_g.at[b]).wait()

    cols_tab = [lax.iota(jnp.int32, 16) + (cc * 16) for cc in range(4)]
    zeros16 = jnp.full((16,), 0, jnp.int32)

    def transpose(b):
        buf = buf_v.at[b]
        tbuf = tbuf_v.at[b]

        def row(j, carry):
            jv = zeros16 + j
            for cc in range(4):
                v = buf[j, pl.ds(cc * 16, 16)]
                plsc.store_scatter(tbuf, [cols_tab[cc], jv], v)
            return carry

        lax.fori_loop(0, 128, row, 0, unroll=4)

    def start_wb(u, b):
        s = u // BB
        bb = u - s * BB
        for cg in range(CG):
            pltpu.async_copy(
                tbuf_v.at[b, pl.ds(cg * 8, 8), pl.ds(0, 128)],
                out_hbm.at[s, cg, bb], s_wb.at[b])

    def wait_wb(b):
        for cg in range(CG):
            pltpu.make_async_copy(
                tbuf_v.at[b, pl.ds(cg * 8, 8), pl.ds(0, 128)],
                out_hbm.at[0, 0, 0], s_wb.at[b]).wait()

    def step(i, k, do_wait_wb, do_fire_idx, do_fire_gather):
        b = k % 4
        wait_gather(b)
        if do_wait_wb:
            wait_wb(b)
        transpose(b)
        start_wb(i, b)
        if do_fire_idx:
            fire_idx(i + 8, k)
        if do_fire_gather:
            wait_idx((k + 3) % 8)
            fire_gather((k + 3) % 8, (k + 3) % 4)

    for m in range(8):
        fire_idx(u0 + m, m)
    for j in range(3):
        wait_idx(j)
        fire_gather(j, j)
    for k in range(8):
        step(u0 + k, k, do_wait_wb=(k >= 4), do_fire_idx=True,
             do_fire_gather=True)

    def octet(q, carry):
        ub = u0 + q * 8
        for k in range(8):
            step(ub + k, k, do_wait_wb=True, do_fire_idx=True,
                 do_fire_gather=True)
        return carry

    lax.fori_loop(1, OCT - 1, octet, 0)

    ub = u0 + (OCT - 1) * 8
    for k in range(8):
        step(ub + k, k, do_wait_wb=True, do_fire_idx=False,
             do_fire_gather=(k <= 4))
    for b in range(4):
        wait_wb(b)


def kernel(x, table):
    xt = x.T
    tp = jnp.pad(table, ((0, 0), (0, 128 - DIM)))
    lout = _gather(xt, tp)
    return lout.transpose(2, 4, 0, 1, 3).reshape(BATCH, SEQ, DIM)


# speedup vs baseline: 1.7046x; 1.0010x over previous
import functools

import jax
import jax.numpy as jnp
from jax import lax
from jax.experimental import pallas as pl
from jax.experimental.pallas import tpu as pltpu
from jax.experimental.pallas import tpu_sc as plsc

VOCAB = 1000000
DIM = 64
BATCH = 16384
SEQ = 50

NC = 2
NS = 16
NW = NC * NS
BB = BATCH // 128
UNITS = SEQ * BB
PER_W = UNITS // NW
CG = DIM // 8
OCT = PER_W // 8

_mesh = plsc.VectorSubcoreMesh(core_axis_name="c", subcore_axis_name="s")


@functools.partial(
    pl.kernel,
    mesh=_mesh,
    compiler_params=pltpu.CompilerParams(
        use_tc_tiling_on_sc=False, needs_layout_passes=False),
    out_type=jax.ShapeDtypeStruct((SEQ, CG, BB, 8, 128), jnp.float32),
    scratch_types=[
        pltpu.VMEM((8, 128), jnp.int32),
        pltpu.VMEM((4, 128, DIM), jnp.float32),
        pltpu.VMEM((4, DIM, 137), jnp.float32),
        pltpu.SemaphoreType.DMA((8,)),
        pltpu.SemaphoreType.DMA((4,)),
        pltpu.SemaphoreType.DMA((4,)),
    ],
)
def _gather(xt_hbm, table_hbm, out_hbm, idx_v, buf_v, tbuf_v, s_idx, s_g, s_wb):
    wid = lax.axis_index("s") * NC + lax.axis_index("c")
    u0 = wid * PER_W

    def fire_idx(u, m):
        s = u // BB
        bb = u - s * BB
        pltpu.async_copy(
            xt_hbm.at[s, pl.ds(pl.multiple_of(bb * 128, 128), 128)],
            idx_v.at[m], s_idx.at[m])

    def wait_idx(m):
        pltpu.make_async_copy(
            xt_hbm.at[0, pl.ds(0, 128)], idx_v.at[m], s_idx.at[m]).wait()

    def fire_gather(m, b):
        pltpu.async_copy(table_hbm.at[idx_v.at[m]], buf_v.at[b], s_g.at[b])

    def wait_gather(b):
        pltpu.make_async_copy(
            table_hbm.at[pl.ds(0, 128)], buf_v.at[b], s_g.at[b]).wait()

    cols_tab = [lax.iota(jnp.int32, 16) + (cc * 16) for cc in range(4)]
    zeros16 = jnp.full((16,), 0, jnp.int32)

    def transpose(b):
        buf = buf_v.at[b]
        tbuf = tbuf_v.at[b]

        def row(j, carry):
            jv = zeros16 + j
            for cc in range(4):
                v = buf[j, pl.ds(cc * 16, 16)]
                plsc.store_scatter(tbuf, [cols_tab[cc], jv], v)
            return carry

        lax.fori_loop(0, 128, row, 0, unroll=4)

    def start_wb(u, b):
        s = u // BB
        bb = u - s * BB
        for cg in range(CG):
            pltpu.async_copy(
                tbuf_v.at[b, pl.ds(cg * 8, 8), pl.ds(0, 128)],
                out_hbm.at[s, cg, bb], s_wb.at[b])

    def wait_wb(b):
        for cg in range(CG):
            pltpu.make_async_copy(
                tbuf_v.at[b, pl.ds(cg * 8, 8), pl.ds(0, 128)],
                out_hbm.at[0, 0, 0], s_wb.at[b]).wait()

    def step(i, k, do_wait_wb, do_fire_idx, do_fire_gather):
        b = k % 4
        wait_gather(b)
        if do_wait_wb:
            wait_wb(b)
        transpose(b)
        start_wb(i, b)
        if do_fire_idx:
            fire_idx(i + 8, k)
        if do_fire_gather:
            wait_idx((k + 3) % 8)
            fire_gather((k + 3) % 8, (k + 3) % 4)

    for m in range(8):
        fire_idx(u0 + m, m)
    for j in range(3):
        wait_idx(j)
        fire_gather(j, j)
    for k in range(8):
        step(u0 + k, k, do_wait_wb=(k >= 4), do_fire_idx=True,
             do_fire_gather=True)

    def octet(q, carry):
        ub = u0 + q * 8
        for k in range(8):
            step(ub + k, k, do_wait_wb=True, do_fire_idx=True,
                 do_fire_gather=True)
        return carry

    lax.fori_loop(1, OCT - 1, octet, 0)

    ub = u0 + (OCT - 1) * 8
    for k in range(8):
        step(ub + k, k, do_wait_wb=True, do_fire_idx=False,
             do_fire_gather=(k <= 4))
    for b in range(4):
        wait_wb(b)


def kernel(x, table):
    xt = (x << 1).T
    tp = jnp.pad(table, ((0, 0), (0, 128 - DIM))).reshape(2 * VOCAB, DIM)
    lout = _gather(xt, tp)
    return lout.transpose(2, 4, 0, 1, 3).reshape(BATCH, SEQ, DIM)


# speedup vs baseline: 1.7106x; 1.0035x over previous
import functools

import jax
import jax.numpy as jnp
from jax import lax
from jax.experimental import pallas as pl
from jax.experimental.pallas import tpu as pltpu
from jax.experimental.pallas import tpu_sc as plsc

VOCAB = 1000000
DIM = 64
BATCH = 16384
SEQ = 50

NC = 2
NS = 16
NW = NC * NS
BB = BATCH // 128
UNITS = SEQ * BB
PER_W = UNITS // NW
CG = DIM // 8
OCT = PER_W // 8

_mesh = plsc.VectorSubcoreMesh(core_axis_name="c", subcore_axis_name="s")


@functools.partial(
    pl.kernel,
    mesh=_mesh,
    compiler_params=pltpu.CompilerParams(
        use_tc_tiling_on_sc=False, needs_layout_passes=False),
    out_type=jax.ShapeDtypeStruct((SEQ, CG, BB * 8, 128), jnp.float32),
    scratch_types=[
        pltpu.VMEM((8, 128), jnp.int32),
        pltpu.VMEM((4, 128, DIM), jnp.float32),
        pltpu.VMEM((4, DIM, 137), jnp.float32),
        pltpu.SemaphoreType.DMA((8,)),
        pltpu.SemaphoreType.DMA((4,)),
        pltpu.SemaphoreType.DMA((4,)),
    ],
)
def _gather(xt_hbm, table_hbm, out_hbm, idx_v, buf_v, tbuf_v, s_idx, s_g, s_wb):
    wid = lax.axis_index("s") * NC + lax.axis_index("c")
    u0 = wid * PER_W

    def fire_idx(u, m):
        s = u // BB
        bb = u - s * BB
        pltpu.async_copy(
            xt_hbm.at[s, pl.ds(pl.multiple_of(bb * 128, 128), 128)],
            idx_v.at[m], s_idx.at[m])

    def wait_idx(m):
        pltpu.make_async_copy(
            xt_hbm.at[0, pl.ds(0, 128)], idx_v.at[m], s_idx.at[m]).wait()

    def fire_gather(m, b):
        pltpu.async_copy(table_hbm.at[idx_v.at[m]], buf_v.at[b], s_g.at[b])

    def wait_gather(b):
        pltpu.make_async_copy(
            table_hbm.at[pl.ds(0, 128)], buf_v.at[b], s_g.at[b]).wait()

    cols_tab = [lax.iota(jnp.int32, 16) + (cc * 16) for cc in range(4)]
    zeros16 = jnp.full((16,), 0, jnp.int32)

    def transpose(b):
        buf = buf_v.at[b]
        tbuf = tbuf_v.at[b]

        def row(j, carry):
            jv = zeros16 + j
            for cc in range(4):
                v = buf[j, pl.ds(cc * 16, 16)]
                plsc.store_scatter(tbuf, [cols_tab[cc], jv], v)
            return carry

        lax.fori_loop(0, 128, row, 0, unroll=4)

    def start_wb(u, b):
        s = u // BB
        bb = u - s * BB
        for cg in range(CG):
            pltpu.async_copy(
                tbuf_v.at[b, pl.ds(cg * 8, 8), pl.ds(0, 128)],
                out_hbm.at[s, cg, pl.ds(bb * 8, 8)], s_wb.at[b])

    def wait_wb(b):
        pltpu.make_async_copy(
            tbuf_v.at[b, pl.ds(0, DIM), pl.ds(0, 128)],
            out_hbm.at[0, 0, pl.ds(0, DIM)], s_wb.at[b]).wait()

    def step(i, k, do_wait_wb, do_fire_idx, do_fire_gather):
        b = k % 4
        wait_gather(b)
        if do_wait_wb:
            wait_wb(b)
        transpose(b)
        start_wb(i, b)
        if do_fire_idx:
            fire_idx(i + 8, k)
        if do_fire_gather:
            wait_idx((k + 3) % 8)
            fire_gather((k + 3) % 8, (k + 3) % 4)

    for m in range(8):
        fire_idx(u0 + m, m)
    for j in range(3):
        wait_idx(j)
        fire_gather(j, j)
    for k in range(8):
        step(u0 + k, k, do_wait_wb=(k >= 4), do_fire_idx=True,
             do_fire_gather=True)

    def octet(q, carry):
        ub = u0 + q * 8
        for k in range(8):
            step(ub + k, k, do_wait_wb=True, do_fire_idx=True,
                 do_fire_gather=True)
        return carry

    lax.fori_loop(1, OCT - 1, octet, 0)

    ub = u0 + (OCT - 1) * 8
    for k in range(8):
        step(ub + k, k, do_wait_wb=True, do_fire_idx=False,
             do_fire_gather=(k <= 4))
    for b in range(4):
        wait_wb(b)


def kernel(x, table):
    xt = (x << 1).T
    tp = jnp.pad(table, ((0, 0), (0, 128 - DIM))).reshape(2 * VOCAB, DIM)
    lout = _gather(xt, tp)
    lout5 = lout.reshape(SEQ, CG, BB, 8, 128)
    return lout5.transpose(2, 4, 0, 1, 3).reshape(BATCH, SEQ, DIM)


# speedup vs baseline: 1.7165x; 1.0034x over previous
import functools

import jax
import jax.numpy as jnp
from jax import lax
from jax.experimental import pallas as pl
from jax.experimental.pallas import tpu as pltpu
from jax.experimental.pallas import tpu_sc as plsc

VOCAB = 1000000
DIM = 64
BATCH = 16384
SEQ = 50

NC = 2
NS = 16
NW = NC * NS
BB = BATCH // 128
UNITS = SEQ * BB
PER_W = UNITS // NW
CG = DIM // 8
OCT = PER_W // 8

_mesh = plsc.VectorSubcoreMesh(core_axis_name="c", subcore_axis_name="s")


@functools.partial(
    pl.kernel,
    mesh=_mesh,
    compiler_params=pltpu.CompilerParams(
        use_tc_tiling_on_sc=False, needs_layout_passes=False),
    out_type=jax.ShapeDtypeStruct((SEQ, CG, BB * 8, 128), jnp.float32),
    scratch_types=[
        pltpu.VMEM((8, 128), jnp.int32),
        pltpu.VMEM((4, 128, DIM), jnp.float32),
        pltpu.VMEM((4, DIM, 137), jnp.float32),
        pltpu.SemaphoreType.DMA((8,)),
        pltpu.SemaphoreType.DMA((4,)),
        pltpu.SemaphoreType.DMA((4,)),
    ],
)
def _gather(xt_hbm, table_hbm, out_hbm, idx_v, buf_v, tbuf_v, s_idx, s_g, s_wb):
    wid = lax.axis_index("s") * NC + lax.axis_index("c")
    u0 = wid * PER_W

    def fire_idx(u, m):
        s = u // BB
        bb = u - s * BB
        pltpu.async_copy(
            xt_hbm.at[s, pl.ds(pl.multiple_of(bb * 128, 128), 128)],
            idx_v.at[m], s_idx.at[m])

    def wait_idx(m):
        pltpu.make_async_copy(
            xt_hbm.at[0, pl.ds(0, 128)], idx_v.at[m], s_idx.at[m]).wait()

    def fire_gather(m, b):
        pltpu.async_copy(table_hbm.at[idx_v.at[m]], buf_v.at[b], s_g.at[b])

    def wait_gather(b):
        pltpu.make_async_copy(
            table_hbm.at[pl.ds(0, 128)], buf_v.at[b], s_g.at[b]).wait()

    cols_tab = [lax.iota(jnp.int32, 16) + (cc * 16) for cc in range(4)]
    zeros16 = jnp.full((16,), 0, jnp.int32)

    def transpose(b):
        buf = buf_v.at[b]
        tbuf = tbuf_v.at[b]

        def row(j, carry):
            jv = zeros16 + j
            for cc in range(4):
                v = buf[j, pl.ds(cc * 16, 16)]
                plsc.store_scatter(tbuf, [cols_tab[cc], jv], v)
            return carry

        lax.fori_loop(0, 128, row, 0, unroll=4)

    def start_wb(u, b):
        s = u // BB
        bb = u - s * BB
        for cg in range(CG):
            pltpu.async_copy(
                tbuf_v.at[b, pl.ds(cg * 8, 8), pl.ds(0, 128)],
                out_hbm.at[s, cg, pl.ds(bb * 8, 8)], s_wb.at[b])

    def wait_wb(b):
        pltpu.make_async_copy(
            tbuf_v.at[b, pl.ds(0, DIM), pl.ds(0, 128)],
            out_hbm.at[0, 0, pl.ds(0, DIM)], s_wb.at[b]).wait()

    for m in range(8):
        fire_idx(u0 + m, m)
    for j in range(3):
        wait_idx(j)
        fire_gather(j, j)

    def octet(q, carry):
        rel = q * 8
        ub = u0 + rel
        for k in range(8):
            i = ub + k
            b = k % 4
            wait_gather(b)
            if k < 4:
                @pl.when(q > 0)
                def _():
                    wait_wb(b)
            else:
                wait_wb(b)
            transpose(b)
            start_wb(i, b)

            @pl.when(rel + k + 8 < PER_W)
            def _():
                fire_idx(i + 8, k)

            @pl.when(rel + k + 3 < PER_W)
            def _():
                wait_idx((k + 3) % 8)
                fire_gather((k + 3) % 8, (k + 3) % 4)
        return carry

    lax.fori_loop(0, OCT, octet, 0)
    for b in range(4):
        wait_wb(b)


def kernel(x, table):
    xt = (x << 1).T
    tp = jnp.pad(table, ((0, 0), (0, 128 - DIM))).reshape(2 * VOCAB, DIM)
    lout = _gather(xt, tp)
    lout5 = lout.reshape(SEQ, CG, BB, 8, 128)
    return lout5.transpose(2, 4, 0, 1, 3).reshape(BATCH, SEQ, DIM)
